# baseline (speedup 1.0000x reference)
import functools

import jax
import jax.numpy as jnp
from jax import lax
from jax.experimental import pallas as pl
from jax.experimental.pallas import tpu as pltpu
from jax.experimental.pallas import tpu_sc as plsc

_CHUNK = 128


@functools.cache
def _build(n_user, n_item, batch, d):
    info = plsc.get_sparse_core_info()
    nw = info.num_cores * info.num_subcores
    nc = info.num_cores
    b_per_w = batch // nw
    n_chunks = b_per_w // _CHUNK
    mesh = plsc.VectorSubcoreMesh(core_axis_name="c", subcore_axis_name="s")

    @functools.partial(
        pl.kernel,
        mesh=mesh,
        out_type=jax.ShapeDtypeStruct((2 * batch, 2 * d), jnp.float32),
        scratch_types=[
            pltpu.VMEM((n_chunks, _CHUNK), jnp.int32),
            pltpu.VMEM((b_per_w, 2 * d), jnp.float32),
            pltpu.SemaphoreType.DMA,
        ],
    )
    def gather_kernel(user_hbm, item_hbm, idx_u_hbm, idx_i_hbm, out_hbm,
                      idx_v, buf, sem):
        wid = lax.axis_index("s") * nc + lax.axis_index("c")
        base = wid * b_per_w
        crow = wid * n_chunks
        for table, idx_hbm, out_base in (
            (user_hbm, idx_u_hbm, base),
            (item_hbm, idx_i_hbm, batch + base),
        ):
            pltpu.sync_copy(idx_hbm.at[pl.ds(crow, n_chunks)], idx_v)
            copies = [
                pltpu.async_copy(table.at[idx_v.at[j]],
                                 buf.at[pl.ds(j * _CHUNK, _CHUNK)], sem)
                for j in range(n_chunks)
            ]
            for c in copies:
                c.wait()
            pltpu.sync_copy(buf, out_hbm.at[pl.ds(out_base, b_per_w)])

    return gather_kernel


@jax.jit
def kernel(embed_user, embed_item, idx_user, idx_item):
    batch = idx_user.shape[0]
    d = embed_user.shape[1]
    idx_user = idx_user.astype(jnp.int32)
    idx_item = idx_item.astype(jnp.int32)
    pair_u = (idx_user >> 1).reshape(batch // _CHUNK, _CHUNK)
    pair_i = (idx_item >> 1).reshape(batch // _CHUNK, _CHUNK)
    k = _build(embed_user.shape[0], embed_item.shape[0], batch, d)
    out_pairs = k(embed_user.reshape(-1, 2 * d), embed_item.reshape(-1, 2 * d),
                  pair_u, pair_i)
    half = jnp.concatenate([idx_user & 1, idx_item & 1])[:, None].astype(bool)
    return jnp.where(half, out_pairs[:, d:], out_pairs[:, :d])
